# Initial kernel scaffold; baseline (speedup 1.0000x reference)
import functools

import jax
import jax.numpy as jnp
from jax import lax
from jax.experimental import pallas as pl
from jax.experimental.pallas import tpu as pltpu
from jax.experimental.pallas import tpu_sc as plsc

_NC = 2
_NS = 16
_NW = _NC * _NS
_CH = 125
_CT = 80
_N = 10000
_D = 128
_RPT = _N // _NS

_GA = -2.0 * 0.7978845608028654
_GB = _GA * 0.044715


def _xl_body(x_ref, w_ref, b_ref, o_ref):
    o_ref[...] = lax.dot_general(
        x_ref[...], w_ref[...], (((1,), (1,)), ((), ())),
        preferred_element_type=jnp.float32) + b_ref[...]


def _xl_call(x, w, b):
    n, d = x.shape
    blk = 2000
    return pl.pallas_call(
        _xl_body,
        grid=(n // blk,),
        in_specs=[
            pl.BlockSpec((blk, d), lambda i: (i, 0)),
            pl.BlockSpec((d, d), lambda i: (0, 0)),
            pl.BlockSpec((1, d), lambda i: (0, 0)),
        ],
        out_specs=pl.BlockSpec((blk, d), lambda i: (i, 0)),
        out_shape=jax.ShapeDtypeStruct((n, d), jnp.float32),
    )(x, w, b)


def _sc_body(xl, src2, dst2, dstq, w2, attr, outp, cntp,
             src_v, dst_v, dstq_v, w_v, g, a, counts, out_sh, sem_g, sem_a):
    cid = lax.axis_index("c")
    sid = lax.axis_index("s")
    wid = sid * _NC + cid

    pltpu.sync_copy(src2.at[pl.ds(wid * _CT, _CT)], src_v)
    pltpu.sync_copy(dst2.at[pl.ds(wid * _CT, _CT)], dst_v)
    pltpu.sync_copy(w2.at[pl.ds(wid * _CT, _CT)], w_v)
    pltpu.sync_copy(dstq.at[pl.ds(wid * _RPT, _RPT)], dstq_v)

    zero16 = jnp.zeros((16,), jnp.float32)

    def _zg(i, c):
        for k in range(8):
            g[i, pl.ds(k * 16, 16)] = zero16
        return c
    lax.fori_loop(0, _CH, _zg, 0)
    for t in range(5):
        pltpu.sync_copy(g, out_sh.at[pl.ds(sid * _RPT + t * _CH, _CH)])

    def _zc(i, c):
        counts[i, :] = zero16
        return c
    lax.fori_loop(0, _RPT, _zc, 0)

    ones16 = jnp.ones((16,), jnp.float32)

    def _cb(q, c):
        dv = dstq_v[q, :]
        plsc.addupdate_scatter(counts, [dv >> 4, dv & 15], ones16)
        return c
    lax.fori_loop(0, _RPT, _cb, 0)

    plsc.subcore_barrier()

    def _chunk(j, c):
        base = wid * _CT + j
        cp_g = pltpu.async_copy(xl.at[src_v.at[j]], g, sem_g)
        cp_a = pltpu.async_copy(attr.at[pl.ds(base * _CH, _CH)], a, sem_a)
        cp_g.wait()
        cp_a.wait()

        def _edge(e, c2):
            wgt = w_v[j, e]
            for k in range(8):
                sl = pl.ds(k * 16, 16)
                xv = g[e, sl] + a[e, sl]
                t = xv * (_GA + _GB * (xv * xv))
                g[e, sl] = (xv * wgt) / (1.0 + jnp.exp(t))
            return c2
        lax.fori_loop(0, _CH, _edge, 0)
        pltpu.sync_copy(g, out_sh.at[dst_v.at[j]], add=True)
        return c
    lax.fori_loop(0, _CT, _chunk, 0)

    plsc.subcore_barrier()

    pltpu.sync_copy(out_sh.at[pl.ds(sid * _RPT, _RPT)],
                    outp.at[cid, pl.ds(sid * _RPT, _RPT)])
    pltpu.sync_copy(counts, cntp.at[wid])


def _sc_call(xl, src2, dst2, dstq, w2, attr):
    mesh = plsc.VectorSubcoreMesh(core_axis_name="c", subcore_axis_name="s")
    f = pl.kernel(
        _sc_body,
        out_type=[
            jax.ShapeDtypeStruct((_NC, _N, _D), jnp.float32),
            jax.ShapeDtypeStruct((_NW, _RPT, 16), jnp.float32),
        ],
        mesh=mesh,
        scratch_types=[
            pltpu.VMEM((_CT, _CH), jnp.int32),
            pltpu.VMEM((_CT, _CH), jnp.int32),
            pltpu.VMEM((_RPT, 16), jnp.int32),
            pltpu.VMEM((_CT, _CH), jnp.float32),
            pltpu.VMEM((_CH, _D), jnp.float32),
            pltpu.VMEM((_CH, _D), jnp.float32),
            pltpu.VMEM((_RPT, 16), jnp.float32),
            pltpu.VMEM_SHARED((_N, _D), jnp.float32),
            pltpu.SemaphoreType.DMA,
            pltpu.SemaphoreType.DMA,
        ],
    )
    return f(xl, src2, dst2, dstq, w2, attr)


def _fin_body(op_ref, cnt_ref, x_ref, wl_ref, bl_ref, wr_ref, o_ref):
    s = op_ref[0] + op_ref[1]
    c = jnp.sum(cnt_ref[...], axis=0)
    r = 1.0 / jnp.maximum(c, 1.0)
    t = lax.dot_general(s, wl_ref[...], (((1,), (1,)), ((), ())),
                        preferred_element_type=jnp.float32)
    u = lax.dot_general(x_ref[...], wr_ref[...], (((1,), (1,)), ((), ())),
                        preferred_element_type=jnp.float32)
    o_ref[...] = t * r[:, None] + bl_ref[...] + u


def _fin_call(outp, cnt, x, wl, bl, wr):
    n, d = x.shape
    blk = 2000
    return pl.pallas_call(
        _fin_body,
        grid=(n // blk,),
        in_specs=[
            pl.BlockSpec((_NC, blk, d), lambda i: (0, i, 0)),
            pl.BlockSpec((_NW, blk), lambda i: (0, i)),
            pl.BlockSpec((blk, d), lambda i: (i, 0)),
            pl.BlockSpec((d, d), lambda i: (0, 0)),
            pl.BlockSpec((1, d), lambda i: (0, 0)),
            pl.BlockSpec((d, d), lambda i: (0, 0)),
        ],
        out_specs=pl.BlockSpec((blk, d), lambda i: (i, 0)),
        out_shape=jax.ShapeDtypeStruct((n, d), jnp.float32),
    )(outp, cnt, x, wl, bl, wr)


def kernel(x, edge_index, edge_attr, edge_weight, W_lin, b_lin, W_l, b_l, W_r):
    n, d = x.shape
    src2 = edge_index[0].astype(jnp.int32).reshape(_NW * _CT, _CH)
    dst2 = edge_index[1].astype(jnp.int32).reshape(_NW * _CT, _CH)
    dstq = edge_index[1].astype(jnp.int32).reshape(_NW * _RPT, 16)
    w2 = edge_weight.reshape(_NW * _CT, _CH)
    xl = _xl_call(x, W_lin, b_lin.reshape(1, d))
    outp, cntp = _sc_call(xl, src2, dst2, dstq, w2, edge_attr)
    cnt = cntp.reshape(_NW, n)
    return _fin_call(outp, cnt, x, W_l, b_l.reshape(1, d), W_r)


# speedup vs baseline: 1.1462x; 1.1462x over previous
import functools

import jax
import jax.numpy as jnp
from jax import lax
from jax.experimental import pallas as pl
from jax.experimental.pallas import tpu as pltpu
from jax.experimental.pallas import tpu_sc as plsc

_NC = 2
_NS = 16
_NW = _NC * _NS
_CH = 80
_CT = 125
_N = 10000
_D = 128
_RPT = _N // _NS

_GA = -2.0 * 0.7978845608028654
_GB = _GA * 0.044715


def _xl_body(x_ref, w_ref, b_ref, o_ref):
    o_ref[...] = lax.dot_general(
        x_ref[...], w_ref[...], (((1,), (1,)), ((), ())),
        preferred_element_type=jnp.float32) + b_ref[...]


def _xl_call(x, w, b):
    n, d = x.shape
    blk = 2000
    return pl.pallas_call(
        _xl_body,
        grid=(n // blk,),
        in_specs=[
            pl.BlockSpec((blk, d), lambda i: (i, 0)),
            pl.BlockSpec((d, d), lambda i: (0, 0)),
            pl.BlockSpec((1, d), lambda i: (0, 0)),
        ],
        out_specs=pl.BlockSpec((blk, d), lambda i: (i, 0)),
        out_shape=jax.ShapeDtypeStruct((n, d), jnp.float32),
    )(x, w, b)


def _sc_body(xl, src1, dst1, w1, attr, outp, cntp,
             src_c, dst_c, dstr_c, w_c, g, a, cbuf, out_sh, cnt_sh,
             sem_g, sem_a):
    cid = lax.axis_index("c")
    sid = lax.axis_index("s")
    wid = sid * _NC + cid

    zero16 = jnp.zeros((16,), jnp.float32)

    def _zg(i, c):
        for k in range(8):
            g[i, pl.ds(k * 16, 16)] = zero16
            cbuf[i, pl.ds(k * 16, 16)] = zero16
        return c
    lax.fori_loop(0, _CH, _zg, 0)
    for t in range(_RPT // _CH):
        pltpu.sync_copy(g, out_sh.at[pl.ds(sid * _RPT + t * _CH, _CH)])
    _rem = _RPT % _CH
    if _rem:
        pltpu.sync_copy(
            g.at[pl.ds(0, _rem)],
            out_sh.at[pl.ds(sid * _RPT + (_RPT // _CH) * _CH, _rem)])
    @pl.when(sid < 10)
    def _zc():
        pltpu.sync_copy(cbuf, cnt_sh.at[pl.ds(sid * 125, _CH)])
        pltpu.sync_copy(cbuf.at[pl.ds(0, 45)],
                        cnt_sh.at[pl.ds(sid * 125 + _CH, 45)])

    plsc.subcore_barrier()

    ones16 = jnp.ones((16,), jnp.float32)

    def _chunk(j, c):
        eoff = (wid * _CT + j) * _CH
        pltpu.sync_copy(src1.at[pl.ds(eoff, _CH)], src_c)
        pltpu.sync_copy(dst1.at[pl.ds(eoff, _CH)], dst_c)
        pltpu.sync_copy(w1.at[pl.ds(eoff, _CH)], w_c)
        for q in range(_CH // 16):
            dv16 = dst_c[pl.ds(q * 16, 16)]
            dstr_c[pl.ds(q * 16, 16)] = dv16 >> 3
        cp_g = pltpu.async_copy(xl.at[src_c], g, sem_g)
        cp_a = pltpu.async_copy(attr.at[pl.ds(eoff, _CH)], a, sem_a)
        cp_g.wait()
        cp_a.wait()

        def _grp(grp, c2):
            e0 = grp * 16
            wv = w_c[pl.ds(e0, 16)]
            dvec = dst_c[pl.ds(e0, 16)]
            for i in range(16):
                e = e0 + i
                wgt = wv[i]
                for k in range(8):
                    sl = pl.ds(k * 16, 16)
                    xv = g[e, sl] + a[e, sl]
                    t = xv * (_GA + _GB * (xv * xv))
                    g[e, sl] = (xv * wgt) / (1.0 + jnp.exp(t))
                off = (dvec[i] & 7) * 16
                cbuf[e, pl.ds(off, 16)] = ones16
            return c2
        lax.fori_loop(0, _CH // 16, _grp, 0)
        pltpu.sync_copy(g, out_sh.at[dst_c], add=True)
        pltpu.sync_copy(cbuf, cnt_sh.at[dstr_c], add=True)

        def _clr(grp, c2):
            e0 = grp * 16
            dvec = dst_c[pl.ds(e0, 16)]
            for i in range(16):
                off = (dvec[i] & 7) * 16
                cbuf[e0 + i, pl.ds(off, 16)] = zero16
            return c2
        lax.fori_loop(0, _CH // 16, _clr, 0)
        return c
    lax.fori_loop(0, _CT, _chunk, 0)

    plsc.subcore_barrier()

    pltpu.sync_copy(out_sh.at[pl.ds(sid * _RPT, _RPT)], outp.at[cid, sid])

    @pl.when(sid == 0)
    def _dc():
        pltpu.sync_copy(cnt_sh, cntp.at[cid])


def _sc_call(xl, src1, dst1, w1, attr):
    mesh = plsc.VectorSubcoreMesh(core_axis_name="c", subcore_axis_name="s")
    f = pl.kernel(
        _sc_body,
        out_type=[
            jax.ShapeDtypeStruct((_NC, _NS, _RPT, _D), jnp.float32),
            jax.ShapeDtypeStruct((_NC, _N // 8, _D), jnp.float32),
        ],
        mesh=mesh,
        scratch_types=[
            pltpu.VMEM((_CH,), jnp.int32),
            pltpu.VMEM((_CH,), jnp.int32),
            pltpu.VMEM((_CH,), jnp.int32),
            pltpu.VMEM((_CH,), jnp.float32),
            pltpu.VMEM((_CH, _D), jnp.float32),
            pltpu.VMEM((_CH, _D), jnp.float32),
            pltpu.VMEM((_CH, _D), jnp.float32),
            pltpu.VMEM_SHARED((_N, _D), jnp.float32),
            pltpu.VMEM_SHARED((_N // 8, _D), jnp.float32),
            pltpu.SemaphoreType.DMA,
            pltpu.SemaphoreType.DMA,
        ],
    )
    return f(xl, src1, dst1, w1, attr)


def _fin_body(op_ref, cnt_ref, x_ref, wl_ref, bl_ref, wr_ref, o_ref):
    s = op_ref[0] + op_ref[1]
    c = cnt_ref[0, 0] + cnt_ref[0, 1]
    r = 1.0 / jnp.maximum(c, 1.0)
    t = lax.dot_general(s, wl_ref[...], (((1,), (1,)), ((), ())),
                        preferred_element_type=jnp.float32)
    u = lax.dot_general(x_ref[...], wr_ref[...], (((1,), (1,)), ((), ())),
                        preferred_element_type=jnp.float32)
    o_ref[...] = t * r[:, None] + bl_ref[...] + u


def _fin_call(outp, cnt, x, wl, bl, wr):
    n, d = x.shape
    blk = 2000
    return pl.pallas_call(
        _fin_body,
        grid=(n // blk,),
        in_specs=[
            pl.BlockSpec((_NC, blk, d), lambda i: (0, i, 0)),
            pl.BlockSpec((1, _NC, blk), lambda i: (i, 0, 0)),
            pl.BlockSpec((blk, d), lambda i: (i, 0)),
            pl.BlockSpec((d, d), lambda i: (0, 0)),
            pl.BlockSpec((1, d), lambda i: (0, 0)),
            pl.BlockSpec((d, d), lambda i: (0, 0)),
        ],
        out_specs=pl.BlockSpec((blk, d), lambda i: (i, 0)),
        out_shape=jax.ShapeDtypeStruct((n, d), jnp.float32),
    )(outp, cnt, x, wl, bl, wr)


def kernel(x, edge_index, edge_attr, edge_weight, W_lin, b_lin, W_l, b_l, W_r):
    n, d = x.shape
    src1 = edge_index[0].astype(jnp.int32)
    dst1 = edge_index[1].astype(jnp.int32)
    w1 = edge_weight.reshape(-1)
    xl = _xl_call(x, W_lin, b_lin.reshape(1, d))
    outp, cntp = _sc_call(xl, src1, dst1, w1, edge_attr)
    cnt = cntp.reshape(_NC, n // 8, 8, 16)[:, :, :, 0].reshape(_NC, 5, n // 5)
    cnt = cnt.transpose(1, 0, 2)
    return _fin_call(outp.reshape(_NC, n, d), cnt, x, W_l, b_l.reshape(1, d),
                     W_r)
